# baseline (speedup 1.0000x reference)
import functools

import jax
import jax.numpy as jnp
from jax import lax
from jax.experimental import pallas as pl
from jax.experimental.pallas import tpu as pltpu
from jax.experimental.pallas import tpu_sc as plsc

B = 2
N = 2048
D = 1024
H = 4096
E = 8
T = B * N
CAP = 320
CAPF = float(CAP)
R = E * B * CAP
EPS = 1e-9
THRESHOLD = 0.2
LOSS_COEF = 0.01

CN = 512
CHUNKS = T // CN
CPB = N // CN
TR, TCOL = 32, 128

HC = 512
HCH = H // HC

NSC, NTEC = 2, 16
NW = NSC * NTEC
RPW = R // NW
RH = RPW // 2
TPW = T // NW
CHK = 32


def _gating_body(x_ref, wg_ref, p_ref, i1_ref, i2_ref, g1_ref, g2_ref,
                 p1_ref, p2_ref, cnt_ref, gs_ref, acc_ref):
    c = pl.program_id(0)

    @pl.when(c % CPB == 0)
    def _():
        acc_ref[...] = jnp.zeros_like(acc_ref)

    xb = x_ref[...]
    logits = jnp.dot(xb, wg_ref[...], preferred_element_type=jnp.float32)
    m = jnp.max(logits, axis=-1, keepdims=True)
    ex = jnp.exp(logits - m)
    raw = ex / jnp.sum(ex, axis=-1, keepdims=True)

    g1 = jnp.max(raw, axis=-1, keepdims=True)
    eids = lax.broadcasted_iota(jnp.int32, (CN, E), 1)
    idx1 = jnp.min(jnp.where(raw == g1, eids, E), axis=-1, keepdims=True)
    mask1 = eids == idx1
    raw2 = jnp.where(mask1, 0.0, raw)
    g2 = jnp.max(raw2, axis=-1, keepdims=True)
    idx2 = jnp.min(jnp.where(raw2 == g2, eids, E), axis=-1, keepdims=True)
    mask2r = eids == idx2

    denom = g1 + g2 + EPS
    g1n = g1 / denom
    g2n = g2 / denom

    pv = p_ref[0]
    keep2 = pv < g2n / jnp.float32(THRESHOLD)
    mask2 = mask2r & keep2

    m1f = mask1.astype(jnp.float32)
    m2f = mask2.astype(jnp.float32)

    ri = lax.broadcasted_iota(jnp.int32, (CN, CN), 0)
    ci = lax.broadcasted_iota(jnp.int32, (CN, CN), 1)
    tril = (ci < ri).astype(jnp.float32)
    ex1 = jnp.dot(tril, m1f, preferred_element_type=jnp.float32) + acc_ref[0:1, 0:E]
    ex2 = jnp.dot(tril, m2f, preferred_element_type=jnp.float32) + acc_ref[1:2, 0:E]
    pos1 = jnp.sum(ex1 * m1f, axis=-1, keepdims=True)
    pos2 = jnp.sum(ex2 * m2f, axis=-1, keepdims=True)
    m2any = jnp.sum(m2f, axis=-1, keepdims=True)

    i1_ref[...] = idx1.reshape(1, CN, 1)
    i2_ref[...] = idx2.reshape(1, CN, 1)
    g1_ref[...] = g1n.reshape(1, CN, 1)
    g2_ref[...] = (g2n * m2any).reshape(1, CN, 1)
    p1_ref[...] = pos1.astype(jnp.int32).reshape(1, CN, 1)
    p2_ref[...] = pos2.astype(jnp.int32).reshape(1, CN, 1)

    acc_ref[0:1, 0:E] += jnp.sum(m1f, axis=0, keepdims=True)
    acc_ref[1:2, 0:E] += jnp.sum(m2f, axis=0, keepdims=True)
    acc_ref[2:3, 0:E] += jnp.sum(raw, axis=0, keepdims=True)

    cnt_ref[...] = acc_ref[0:1, 0:E].reshape(1, 1, E)
    gs_ref[...] = acc_ref[2:3, 0:E].reshape(1, 1, E)


def _run_gating(x2d, w_gating, probs3):
    tok3 = jax.ShapeDtypeStruct((CHUNKS, CN, 1), jnp.int32)
    tok3f = jax.ShapeDtypeStruct((CHUNKS, CN, 1), jnp.float32)
    be = jax.ShapeDtypeStruct((B, 1, E), jnp.float32)
    return pl.pallas_call(
        _gating_body,
        grid=(CHUNKS,),
        in_specs=[
            pl.BlockSpec((CN, D), lambda c: (c, 0)),
            pl.BlockSpec((D, E), lambda c: (0, 0)),
            pl.BlockSpec((1, CN, 1), lambda c: (c, 0, 0)),
        ],
        out_specs=[
            pl.BlockSpec((1, CN, 1), lambda c: (c, 0, 0)),
            pl.BlockSpec((1, CN, 1), lambda c: (c, 0, 0)),
            pl.BlockSpec((1, CN, 1), lambda c: (c, 0, 0)),
            pl.BlockSpec((1, CN, 1), lambda c: (c, 0, 0)),
            pl.BlockSpec((1, CN, 1), lambda c: (c, 0, 0)),
            pl.BlockSpec((1, CN, 1), lambda c: (c, 0, 0)),
            pl.BlockSpec((1, 1, E), lambda c: (c // CPB, 0, 0)),
            pl.BlockSpec((1, 1, E), lambda c: (c // CPB, 0, 0)),
        ],
        out_shape=[tok3, tok3, tok3f, tok3f, tok3, tok3, be, be],
        scratch_shapes=[pltpu.VMEM((8, 128), jnp.float32)],
    )(x2d, w_gating, probs3)


def _finalize_body(i1_ref, i2_ref, p1_ref, p2r_ref, g1_ref, g2_ref,
                   cnt_ref, gs_ref, r1_ref, r2_ref, s1_ref, s2_ref, loss_ref):
    i1 = i1_ref[...]
    i2 = i2_ref[...]
    p1 = p1_ref[...]
    p2r = p2r_ref[...].astype(jnp.float32)
    g1 = g1_ref[...]
    g2e = g2_ref[...]
    rows = lax.broadcasted_iota(jnp.int32, (TR, TCOL), 0)
    cols = lax.broadcasted_iota(jnp.int32, (TR, TCOL), 1)
    tok = rows * TCOL + cols
    b = (tok >= N).astype(jnp.int32)
    flat2 = b * E + i2
    add = jnp.zeros((TR, TCOL), jnp.float32)
    for j in range(B * E):
        mj = jnp.minimum(cnt_ref[j // E, 0, j % E], CAPF)
        add = jnp.where(flat2 == j, mj, add)
    pos2 = p2r + add
    kept1 = p1 < CAP
    kept2 = (g2e > 0.0) & (pos2 < CAPF)
    s1 = g1 * kept1.astype(jnp.float32)
    s2 = g2e * kept2.astype(jnp.float32)
    pos1c = jnp.minimum(p1, CAP - 1)
    pos2c = jnp.minimum(pos2.astype(jnp.int32), CAP - 1)
    r1_ref[...] = i1 * (B * CAP) + b * CAP + pos1c
    r2_ref[...] = i2 * (B * CAP) + b * CAP + pos2c
    s1_ref[...] = s1
    s2_ref[...] = s2
    lv = jnp.float32(0.0)
    for j in range(B * E):
        lv = lv + gs_ref[j // E, 0, j % E] * cnt_ref[j // E, 0, j % E]
    loss_ref[0, 0] = lv * jnp.float32((E * E) * LOSS_COEF / (B * E * N * N))


def _run_finalize(i1, i2, p1, p2r, g1, g2e, cnt, gs):
    tk = jax.ShapeDtypeStruct((TR, TCOL), jnp.int32)
    tkf = jax.ShapeDtypeStruct((TR, TCOL), jnp.float32)
    return pl.pallas_call(
        _finalize_body,
        in_specs=[
            pl.BlockSpec((TR, TCOL), lambda: (0, 0)),
            pl.BlockSpec((TR, TCOL), lambda: (0, 0)),
            pl.BlockSpec((TR, TCOL), lambda: (0, 0)),
            pl.BlockSpec((TR, TCOL), lambda: (0, 0)),
            pl.BlockSpec((TR, TCOL), lambda: (0, 0)),
            pl.BlockSpec((TR, TCOL), lambda: (0, 0)),
            pl.BlockSpec(memory_space=pltpu.SMEM),
            pl.BlockSpec(memory_space=pltpu.SMEM),
        ],
        out_specs=[
            pl.BlockSpec((TR, TCOL), lambda: (0, 0)),
            pl.BlockSpec((TR, TCOL), lambda: (0, 0)),
            pl.BlockSpec((TR, TCOL), lambda: (0, 0)),
            pl.BlockSpec((TR, TCOL), lambda: (0, 0)),
            pl.BlockSpec(memory_space=pltpu.SMEM),
        ],
        out_shape=[tk, tk, tkf, tkf,
                   jax.ShapeDtypeStruct((1, 1), jnp.float32)],
    )(i1, i2, p1, p2r, g1, g2e, cnt, gs)


@functools.lru_cache(maxsize=None)
def _sc_mesh():
    return plsc.VectorSubcoreMesh(
        core_axis_name="c", subcore_axis_name="s",
        num_cores=NSC, num_subcores=NTEC)


def _sc_build_table_body(r1_hbm, r2_hbm, s1_hbm, s2_hbm, zero_hbm, table_hbm,
                         tab_v, r1_v, r2_v, s1_v, s2_v):
    wid = lax.axis_index("s") * NSC + lax.axis_index("c")

    @pl.when(wid == 0)
    def _():
        pltpu.sync_copy(zero_hbm, tab_v)
        pltpu.sync_copy(r1_hbm, r1_v)
        pltpu.sync_copy(r2_hbm, r2_v)
        pltpu.sync_copy(s1_hbm, s1_v)
        pltpu.sync_copy(s2_hbm, s2_v)
        lanes = lax.iota(jnp.int32, 16)

        def body(i, carry):
            base16 = i * 16
            t = lanes + base16
            plsc.store_scatter(tab_v, [r1_v[pl.ds(base16, 16)]], t,
                               mask=s1_v[pl.ds(base16, 16)] > 0.0)
            plsc.store_scatter(tab_v, [r2_v[pl.ds(base16, 16)]], t,
                               mask=s2_v[pl.ds(base16, 16)] > 0.0)
            return carry

        lax.fori_loop(0, T // 16, body, 0)
        pltpu.sync_copy(tab_v, table_hbm)


@functools.lru_cache(maxsize=None)
def _sc_build_table():
    return pl.kernel(
        _sc_build_table_body,
        out_type=jax.ShapeDtypeStruct((R,), jnp.int32),
        mesh=_sc_mesh(),
        compiler_params=pltpu.CompilerParams(needs_layout_passes=False),
        scratch_types=[
            pltpu.VMEM((R,), jnp.int32),
            pltpu.VMEM((T,), jnp.int32),
            pltpu.VMEM((T,), jnp.int32),
            pltpu.VMEM((T,), jnp.float32),
            pltpu.VMEM((T,), jnp.float32),
        ],
    )


def _sc_dispatch_body(x_hbm, src_hbm, ei_hbm, idx_v, rows_v, sem):
    wid = lax.axis_index("s") * NSC + lax.axis_index("c")
    base = wid * RPW
    pltpu.sync_copy(src_hbm.at[pl.ds(base, RPW)], idx_v)
    for hh in range(RPW // RH):
        pltpu.async_copy(
            x_hbm.at[idx_v.at[pl.ds(hh * RH, RH)]], rows_v, sem).wait()
        pltpu.sync_copy(rows_v, ei_hbm.at[pl.ds(base + hh * RH, RH)])


@functools.lru_cache(maxsize=None)
def _sc_dispatch():
    return pl.kernel(
        _sc_dispatch_body,
        out_type=jax.ShapeDtypeStruct((R, D), jnp.float32),
        mesh=_sc_mesh(),
        compiler_params=pltpu.CompilerParams(needs_layout_passes=False),
        scratch_types=[
            pltpu.VMEM((RPW,), jnp.int32),
            pltpu.VMEM((RH, D), jnp.float32),
            pltpu.SemaphoreType.DMA,
        ],
    )


def _ffn_body(ei_ref, w1_ref, w2_ref, out_ref, acc_ref):
    h = pl.program_id(1)
    hid = jnp.dot(ei_ref[0], w1_ref[0], preferred_element_type=jnp.float32)
    hid = jnp.where(hid >= 0.0, hid, hid * jnp.float32(0.01))
    part = jnp.dot(hid, w2_ref[0], preferred_element_type=jnp.float32)

    @pl.when(h == 0)
    def _():
        acc_ref[...] = part

    @pl.when(h > 0)
    def _():
        acc_ref[...] += part

    @pl.when(h == HCH - 1)
    def _():
        out_ref[0] = acc_ref[...]


def _run_ffn(ei3, w1, w2):
    bc = B * CAP
    return pl.pallas_call(
        _ffn_body,
        grid=(E, HCH),
        in_specs=[
            pl.BlockSpec((1, bc, D), lambda e, h: (e, 0, 0)),
            pl.BlockSpec((1, D, HC), lambda e, h: (e, 0, h)),
            pl.BlockSpec((1, HC, D), lambda e, h: (e, h, 0)),
        ],
        out_specs=pl.BlockSpec((1, bc, D), lambda e, h: (e, 0, 0)),
        out_shape=jax.ShapeDtypeStruct((E, bc, D), jnp.float32),
        scratch_shapes=[pltpu.VMEM((bc, D), jnp.float32)],
    )(ei3, w1, w2)


def _sc_combine_body(eo_hbm, r1_hbm, r2_hbm, s1_hbm, s2_hbm, out_hbm,
                     r1_v, r2_v, s1_v, s2_v, a_v, b_v, o_v, sem1, sem2):
    wid = lax.axis_index("s") * NSC + lax.axis_index("c")
    base = wid * TPW
    pltpu.sync_copy(r1_hbm.at[pl.ds(base, TPW)], r1_v)
    pltpu.sync_copy(r2_hbm.at[pl.ds(base, TPW)], r2_v)
    pltpu.sync_copy(s1_hbm.at[pl.ds(base, TPW)], s1_v)
    pltpu.sync_copy(s2_hbm.at[pl.ds(base, TPW)], s2_v)
    for cc in range(TPW // CHK):
        cp1 = pltpu.async_copy(
            eo_hbm.at[r1_v.at[pl.ds(cc * CHK, CHK)]], a_v, sem1)
        cp2 = pltpu.async_copy(
            eo_hbm.at[r2_v.at[pl.ds(cc * CHK, CHK)]], b_v, sem2)
        cp1.wait()
        cp2.wait()

        def tok(j, carry):
            jj = cc * CHK + j
            sidx = jnp.zeros((16,), jnp.int32) + jj
            s1b = plsc.load_gather(s1_v, [sidx])
            s2b = plsc.load_gather(s2_v, [sidx])
            for v in range(D // 16):
                av = a_v[j, pl.ds(v * 16, 16)]
                bv = b_v[j, pl.ds(v * 16, 16)]
                o_v[j, pl.ds(v * 16, 16)] = s1b * av + s2b * bv
            return carry

        lax.fori_loop(0, CHK, tok, 0)
        pltpu.sync_copy(o_v, out_hbm.at[pl.ds(base + cc * CHK, CHK)])


@functools.lru_cache(maxsize=None)
def _sc_combine():
    return pl.kernel(
        _sc_combine_body,
        out_type=jax.ShapeDtypeStruct((T, D), jnp.float32),
        mesh=_sc_mesh(),
        compiler_params=pltpu.CompilerParams(needs_layout_passes=False),
        scratch_types=[
            pltpu.VMEM((TPW,), jnp.int32),
            pltpu.VMEM((TPW,), jnp.int32),
            pltpu.VMEM((TPW,), jnp.float32),
            pltpu.VMEM((TPW,), jnp.float32),
            pltpu.VMEM((CHK, D), jnp.float32),
            pltpu.VMEM((CHK, D), jnp.float32),
            pltpu.VMEM((CHK, D), jnp.float32),
            pltpu.SemaphoreType.DMA,
            pltpu.SemaphoreType.DMA,
        ],
    )


def kernel(x, w_gating, w1, w2, probs):
    x2d = x.reshape(T, D)
    probs3 = probs.reshape(CHUNKS, CN, 1)
    i1, i2, g1n, g2e, p1, p2r, cnt, gs = _run_gating(x2d, w_gating, probs3)
    row1, row2, s1, s2, loss11 = _run_finalize(
        i1.reshape(TR, TCOL), i2.reshape(TR, TCOL),
        p1.reshape(TR, TCOL), p2r.reshape(TR, TCOL),
        g1n.reshape(TR, TCOL), g2e.reshape(TR, TCOL), cnt, gs)
    r1f = row1.reshape(T)
    r2f = row2.reshape(T)
    s1f = s1.reshape(T)
    s2f = s2.reshape(T)
    slot_src = _sc_build_table()(r1f, r2f, s1f, s2f, jnp.zeros((R,), jnp.int32))
    ei = _sc_dispatch()(x2d, slot_src)
    eo = _run_ffn(ei.reshape(E, B * CAP, D), w1, w2)
    out = _sc_combine()(eo.reshape(R, D), r1f, r2f, s1f, s2f)
    return out.reshape(B, N, D), loss11.reshape(())


# speedup vs baseline: 1.0050x; 1.0050x over previous
import functools

import jax
import jax.numpy as jnp
from jax import lax
from jax.experimental import pallas as pl
from jax.experimental.pallas import tpu as pltpu
from jax.experimental.pallas import tpu_sc as plsc

B = 2
N = 2048
D = 1024
H = 4096
E = 8
T = B * N
CAP = 320
CAPF = float(CAP)
R = E * B * CAP
EPS = 1e-9
THRESHOLD = 0.2
LOSS_COEF = 0.01

CN = 512
CHUNKS = T // CN
CPB = N // CN
TR, TCOL = 32, 128

HC = 512
HCH = H // HC

NSC, NTEC = 2, 16
NW = NSC * NTEC
RPW = R // NW
RH = RPW // 2
TPW = T // NW
CHK = 32


def _gating_body(x_ref, wg_ref, p_ref, i1_ref, i2_ref, g1_ref, g2_ref,
                 p1_ref, p2_ref, cnt_ref, gs_ref, acc_ref):
    c = pl.program_id(0)

    @pl.when(c % CPB == 0)
    def _():
        acc_ref[...] = jnp.zeros_like(acc_ref)

    xb = x_ref[...]
    logits = jnp.dot(xb, wg_ref[...], preferred_element_type=jnp.float32)
    m = jnp.max(logits, axis=-1, keepdims=True)
    ex = jnp.exp(logits - m)
    raw = ex / jnp.sum(ex, axis=-1, keepdims=True)

    g1 = jnp.max(raw, axis=-1, keepdims=True)
    eids = lax.broadcasted_iota(jnp.int32, (CN, E), 1)
    idx1 = jnp.min(jnp.where(raw == g1, eids, E), axis=-1, keepdims=True)
    mask1 = eids == idx1
    raw2 = jnp.where(mask1, 0.0, raw)
    g2 = jnp.max(raw2, axis=-1, keepdims=True)
    idx2 = jnp.min(jnp.where(raw2 == g2, eids, E), axis=-1, keepdims=True)
    mask2r = eids == idx2

    denom = g1 + g2 + EPS
    g1n = g1 / denom
    g2n = g2 / denom

    pv = p_ref[0]
    keep2 = pv < g2n / jnp.float32(THRESHOLD)
    mask2 = mask2r & keep2

    m1f = mask1.astype(jnp.float32)
    m2f = mask2.astype(jnp.float32)

    ri = lax.broadcasted_iota(jnp.int32, (CN, CN), 0)
    ci = lax.broadcasted_iota(jnp.int32, (CN, CN), 1)
    tril = (ci < ri).astype(jnp.float32)
    ex1 = jnp.dot(tril, m1f, preferred_element_type=jnp.float32) + acc_ref[0:1, 0:E]
    ex2 = jnp.dot(tril, m2f, preferred_element_type=jnp.float32) + acc_ref[1:2, 0:E]
    pos1 = jnp.sum(ex1 * m1f, axis=-1, keepdims=True)
    pos2 = jnp.sum(ex2 * m2f, axis=-1, keepdims=True)
    m2any = jnp.sum(m2f, axis=-1, keepdims=True)

    i1_ref[...] = idx1.reshape(1, CN, 1)
    i2_ref[...] = idx2.reshape(1, CN, 1)
    g1_ref[...] = g1n.reshape(1, CN, 1)
    g2_ref[...] = (g2n * m2any).reshape(1, CN, 1)
    p1_ref[...] = pos1.astype(jnp.int32).reshape(1, CN, 1)
    p2_ref[...] = pos2.astype(jnp.int32).reshape(1, CN, 1)

    acc_ref[0:1, 0:E] += jnp.sum(m1f, axis=0, keepdims=True)
    acc_ref[1:2, 0:E] += jnp.sum(m2f, axis=0, keepdims=True)
    acc_ref[2:3, 0:E] += jnp.sum(raw, axis=0, keepdims=True)

    cnt_ref[...] = acc_ref[0:1, 0:E].reshape(1, 1, E)
    gs_ref[...] = acc_ref[2:3, 0:E].reshape(1, 1, E)


def _run_gating(x2d, w_gating, probs3):
    tok3 = jax.ShapeDtypeStruct((CHUNKS, CN, 1), jnp.int32)
    tok3f = jax.ShapeDtypeStruct((CHUNKS, CN, 1), jnp.float32)
    be = jax.ShapeDtypeStruct((B, 1, E), jnp.float32)
    return pl.pallas_call(
        _gating_body,
        grid=(CHUNKS,),
        in_specs=[
            pl.BlockSpec((CN, D), lambda c: (c, 0)),
            pl.BlockSpec((D, E), lambda c: (0, 0)),
            pl.BlockSpec((1, CN, 1), lambda c: (c, 0, 0)),
        ],
        out_specs=[
            pl.BlockSpec((1, CN, 1), lambda c: (c, 0, 0)),
            pl.BlockSpec((1, CN, 1), lambda c: (c, 0, 0)),
            pl.BlockSpec((1, CN, 1), lambda c: (c, 0, 0)),
            pl.BlockSpec((1, CN, 1), lambda c: (c, 0, 0)),
            pl.BlockSpec((1, CN, 1), lambda c: (c, 0, 0)),
            pl.BlockSpec((1, CN, 1), lambda c: (c, 0, 0)),
            pl.BlockSpec((1, 1, E), lambda c: (c // CPB, 0, 0)),
            pl.BlockSpec((1, 1, E), lambda c: (c // CPB, 0, 0)),
        ],
        out_shape=[tok3, tok3, tok3f, tok3f, tok3, tok3, be, be],
        scratch_shapes=[pltpu.VMEM((8, 128), jnp.float32)],
    )(x2d, w_gating, probs3)


def _finalize_body(i1_ref, i2_ref, p1_ref, p2r_ref, g1_ref, g2_ref,
                   cnt_ref, gs_ref, r1_ref, r2_ref, s1_ref, s2_ref, loss_ref):
    i1 = i1_ref[...]
    i2 = i2_ref[...]
    p1 = p1_ref[...]
    p2r = p2r_ref[...].astype(jnp.float32)
    g1 = g1_ref[...]
    g2e = g2_ref[...]
    rows = lax.broadcasted_iota(jnp.int32, (TR, TCOL), 0)
    cols = lax.broadcasted_iota(jnp.int32, (TR, TCOL), 1)
    tok = rows * TCOL + cols
    b = (tok >= N).astype(jnp.int32)
    flat2 = b * E + i2
    add = jnp.zeros((TR, TCOL), jnp.float32)
    for j in range(B * E):
        mj = jnp.minimum(cnt_ref[j // E, 0, j % E], CAPF)
        add = jnp.where(flat2 == j, mj, add)
    pos2 = p2r + add
    kept1 = p1 < CAP
    kept2 = (g2e > 0.0) & (pos2 < CAPF)
    s1 = g1 * kept1.astype(jnp.float32)
    s2 = g2e * kept2.astype(jnp.float32)
    pos1c = jnp.minimum(p1, CAP - 1)
    pos2c = jnp.minimum(pos2.astype(jnp.int32), CAP - 1)
    r1_ref[...] = i1 * (B * CAP) + b * CAP + pos1c
    r2_ref[...] = i2 * (B * CAP) + b * CAP + pos2c
    s1_ref[...] = s1
    s2_ref[...] = s2
    lv = jnp.float32(0.0)
    for j in range(B * E):
        lv = lv + gs_ref[j // E, 0, j % E] * cnt_ref[j // E, 0, j % E]
    loss_ref[0, 0] = lv * jnp.float32((E * E) * LOSS_COEF / (B * E * N * N))


def _run_finalize(i1, i2, p1, p2r, g1, g2e, cnt, gs):
    tk = jax.ShapeDtypeStruct((TR, TCOL), jnp.int32)
    tkf = jax.ShapeDtypeStruct((TR, TCOL), jnp.float32)
    return pl.pallas_call(
        _finalize_body,
        in_specs=[
            pl.BlockSpec((TR, TCOL), lambda: (0, 0)),
            pl.BlockSpec((TR, TCOL), lambda: (0, 0)),
            pl.BlockSpec((TR, TCOL), lambda: (0, 0)),
            pl.BlockSpec((TR, TCOL), lambda: (0, 0)),
            pl.BlockSpec((TR, TCOL), lambda: (0, 0)),
            pl.BlockSpec((TR, TCOL), lambda: (0, 0)),
            pl.BlockSpec(memory_space=pltpu.SMEM),
            pl.BlockSpec(memory_space=pltpu.SMEM),
        ],
        out_specs=[
            pl.BlockSpec((TR, TCOL), lambda: (0, 0)),
            pl.BlockSpec((TR, TCOL), lambda: (0, 0)),
            pl.BlockSpec((TR, TCOL), lambda: (0, 0)),
            pl.BlockSpec((TR, TCOL), lambda: (0, 0)),
            pl.BlockSpec(memory_space=pltpu.SMEM),
        ],
        out_shape=[tk, tk, tkf, tkf,
                   jax.ShapeDtypeStruct((1, 1), jnp.float32)],
    )(i1, i2, p1, p2r, g1, g2e, cnt, gs)


@functools.lru_cache(maxsize=None)
def _sc_mesh():
    return plsc.VectorSubcoreMesh(
        core_axis_name="c", subcore_axis_name="s",
        num_cores=NSC, num_subcores=NTEC)


def _sc_build_table_body(r1_hbm, r2_hbm, s1_hbm, s2_hbm, zero_hbm, table_hbm,
                         tab_v, r1_v, r2_v, s1_v, s2_v):
    wid = lax.axis_index("s") * NSC + lax.axis_index("c")

    @pl.when(wid == 0)
    def _():
        pltpu.sync_copy(zero_hbm, tab_v)
        pltpu.sync_copy(r1_hbm, r1_v)
        pltpu.sync_copy(r2_hbm, r2_v)
        pltpu.sync_copy(s1_hbm, s1_v)
        pltpu.sync_copy(s2_hbm, s2_v)
        lanes = lax.iota(jnp.int32, 16)

        def body(i, carry):
            base16 = i * 16
            t = lanes + base16
            plsc.store_scatter(tab_v, [r1_v[pl.ds(base16, 16)]], t,
                               mask=s1_v[pl.ds(base16, 16)] > 0.0)
            plsc.store_scatter(tab_v, [r2_v[pl.ds(base16, 16)]], t,
                               mask=s2_v[pl.ds(base16, 16)] > 0.0)
            return carry

        lax.fori_loop(0, T // 16, body, 0)
        pltpu.sync_copy(tab_v, table_hbm)


@functools.lru_cache(maxsize=None)
def _sc_build_table():
    return pl.kernel(
        _sc_build_table_body,
        out_type=jax.ShapeDtypeStruct((R,), jnp.int32),
        mesh=_sc_mesh(),
        compiler_params=pltpu.CompilerParams(needs_layout_passes=False),
        scratch_types=[
            pltpu.VMEM((R,), jnp.int32),
            pltpu.VMEM((T,), jnp.int32),
            pltpu.VMEM((T,), jnp.int32),
            pltpu.VMEM((T,), jnp.float32),
            pltpu.VMEM((T,), jnp.float32),
        ],
    )


def _sc_dispatch_body(x_hbm, src_hbm, ei_hbm, idx_v, rows_v, sem):
    wid = lax.axis_index("s") * NSC + lax.axis_index("c")
    base = wid * RPW
    pltpu.sync_copy(src_hbm.at[pl.ds(base, RPW)], idx_v)
    for hh in range(RPW // RH):
        pltpu.async_copy(
            x_hbm.at[idx_v.at[pl.ds(hh * RH, RH)]], rows_v, sem).wait()
        pltpu.sync_copy(rows_v, ei_hbm.at[pl.ds(base + hh * RH, RH)])


@functools.lru_cache(maxsize=None)
def _sc_dispatch():
    return pl.kernel(
        _sc_dispatch_body,
        out_type=jax.ShapeDtypeStruct((R, D), jnp.float32),
        mesh=_sc_mesh(),
        compiler_params=pltpu.CompilerParams(needs_layout_passes=False),
        scratch_types=[
            pltpu.VMEM((RPW,), jnp.int32),
            pltpu.VMEM((RH, D), jnp.float32),
            pltpu.SemaphoreType.DMA,
        ],
    )


def _ffn_body(ei_ref, w1_ref, w2_ref, out_ref, acc_ref):
    h = pl.program_id(1)
    ei = ei_ref[0].astype(jnp.bfloat16)
    hid = jnp.dot(ei, w1_ref[0].astype(jnp.bfloat16),
                  preferred_element_type=jnp.float32)
    hid = jnp.where(hid >= 0.0, hid, hid * jnp.float32(0.01))
    part = jnp.dot(hid.astype(jnp.bfloat16), w2_ref[0].astype(jnp.bfloat16),
                   preferred_element_type=jnp.float32)

    @pl.when(h == 0)
    def _():
        acc_ref[...] = part

    @pl.when(h > 0)
    def _():
        acc_ref[...] += part

    @pl.when(h == HCH - 1)
    def _():
        out_ref[0] = acc_ref[...]


def _run_ffn(ei3, w1, w2):
    bc = B * CAP
    return pl.pallas_call(
        _ffn_body,
        grid=(E, HCH),
        in_specs=[
            pl.BlockSpec((1, bc, D), lambda e, h: (e, 0, 0)),
            pl.BlockSpec((1, D, HC), lambda e, h: (e, 0, h)),
            pl.BlockSpec((1, HC, D), lambda e, h: (e, h, 0)),
        ],
        out_specs=pl.BlockSpec((1, bc, D), lambda e, h: (e, 0, 0)),
        out_shape=jax.ShapeDtypeStruct((E, bc, D), jnp.float32),
        scratch_shapes=[pltpu.VMEM((bc, D), jnp.float32)],
    )(ei3, w1, w2)


def _sc_combine_body(eo_hbm, r1_hbm, r2_hbm, s1_hbm, s2_hbm, out_hbm,
                     r1_v, r2_v, s1_v, s2_v, a_v, b_v, o_v, sem1, sem2):
    wid = lax.axis_index("s") * NSC + lax.axis_index("c")
    base = wid * TPW
    pltpu.sync_copy(r1_hbm.at[pl.ds(base, TPW)], r1_v)
    pltpu.sync_copy(r2_hbm.at[pl.ds(base, TPW)], r2_v)
    pltpu.sync_copy(s1_hbm.at[pl.ds(base, TPW)], s1_v)
    pltpu.sync_copy(s2_hbm.at[pl.ds(base, TPW)], s2_v)
    for cc in range(TPW // CHK):
        cp1 = pltpu.async_copy(
            eo_hbm.at[r1_v.at[pl.ds(cc * CHK, CHK)]], a_v, sem1)
        cp2 = pltpu.async_copy(
            eo_hbm.at[r2_v.at[pl.ds(cc * CHK, CHK)]], b_v, sem2)
        cp1.wait()
        cp2.wait()

        def tok(j, carry):
            jj = cc * CHK + j
            sidx = jnp.zeros((16,), jnp.int32) + jj
            s1b = plsc.load_gather(s1_v, [sidx])
            s2b = plsc.load_gather(s2_v, [sidx])
            for v in range(D // 16):
                av = a_v[j, pl.ds(v * 16, 16)]
                bv = b_v[j, pl.ds(v * 16, 16)]
                o_v[j, pl.ds(v * 16, 16)] = s1b * av + s2b * bv
            return carry

        lax.fori_loop(0, CHK, tok, 0)
        pltpu.sync_copy(o_v, out_hbm.at[pl.ds(base + cc * CHK, CHK)])


@functools.lru_cache(maxsize=None)
def _sc_combine():
    return pl.kernel(
        _sc_combine_body,
        out_type=jax.ShapeDtypeStruct((T, D), jnp.float32),
        mesh=_sc_mesh(),
        compiler_params=pltpu.CompilerParams(needs_layout_passes=False),
        scratch_types=[
            pltpu.VMEM((TPW,), jnp.int32),
            pltpu.VMEM((TPW,), jnp.int32),
            pltpu.VMEM((TPW,), jnp.float32),
            pltpu.VMEM((TPW,), jnp.float32),
            pltpu.VMEM((CHK, D), jnp.float32),
            pltpu.VMEM((CHK, D), jnp.float32),
            pltpu.VMEM((CHK, D), jnp.float32),
            pltpu.SemaphoreType.DMA,
            pltpu.SemaphoreType.DMA,
        ],
    )


def kernel(x, w_gating, w1, w2, probs):
    x2d = x.reshape(T, D)
    probs3 = probs.reshape(CHUNKS, CN, 1)
    i1, i2, g1n, g2e, p1, p2r, cnt, gs = _run_gating(x2d, w_gating, probs3)
    row1, row2, s1, s2, loss11 = _run_finalize(
        i1.reshape(TR, TCOL), i2.reshape(TR, TCOL),
        p1.reshape(TR, TCOL), p2r.reshape(TR, TCOL),
        g1n.reshape(TR, TCOL), g2e.reshape(TR, TCOL), cnt, gs)
    r1f = row1.reshape(T)
    r2f = row2.reshape(T)
    s1f = s1.reshape(T)
    s2f = s2.reshape(T)
    slot_src = _sc_build_table()(r1f, r2f, s1f, s2f, jnp.zeros((R,), jnp.int32))
    ei = _sc_dispatch()(x2d, slot_src)
    eo = _run_ffn(ei.reshape(E, B * CAP, D), w1, w2)
    out = _sc_combine()(eo.reshape(R, D), r1f, r2f, s1f, s2f)
    return out.reshape(B, N, D), loss11.reshape(())
